# Initial kernel scaffold; baseline (speedup 1.0000x reference)
import jax
import jax.numpy as jnp
from jax.experimental import pallas as pl


def kernel(inputs, embedding):
    raise NotImplementedError("write your pallas kernel here")


# speedup vs baseline: 1.5463x; 1.5463x over previous
import functools

import jax
import jax.numpy as jnp
from jax import lax
from jax.experimental import pallas as pl
from jax.experimental.pallas import tpu as pltpu
from jax.experimental.pallas import tpu_sc as plsc

_VOCAB = 1000000
_EMBED_DIM = 32
_BATCH = 16384
_FIELDS = 26

_N_TOTAL = _BATCH * _FIELDS
_NC, _NS = 2, 16
_NW = _NC * _NS
_PER_W = _N_TOTAL // _NW
_CHUNK = 1024
_N_CHUNKS = _PER_W // _CHUNK

_mesh = plsc.VectorSubcoreMesh(core_axis_name="c", subcore_axis_name="s")


@functools.partial(
    pl.kernel,
    mesh=_mesh,
    out_type=jax.ShapeDtypeStruct((_N_TOTAL, _EMBED_DIM), jnp.float32),
    scratch_types=[
        pltpu.VMEM((_CHUNK,), jnp.int32),
        pltpu.VMEM((_CHUNK, _EMBED_DIM), jnp.float32),
        pltpu.SemaphoreType.DMA,
    ],
    compiler_params=pltpu.CompilerParams(use_tc_tiling_on_sc=False),
)
def _gather_kernel(idx_hbm, table_hbm, out_hbm, idx_v, rows_v, sem):
    wid = lax.axis_index("s") * _NC + lax.axis_index("c")
    base = wid * _PER_W

    def body(c, carry):
        off = base + c * _CHUNK
        pltpu.sync_copy(idx_hbm.at[pl.ds(off, _CHUNK)], idx_v)
        pltpu.async_copy(table_hbm.at[idx_v], rows_v, sem).wait()
        pltpu.sync_copy(rows_v, out_hbm.at[pl.ds(off, _CHUNK)])
        return carry

    lax.fori_loop(0, _N_CHUNKS, body, 0)


def kernel(inputs, embedding):
    flat_idx = inputs.reshape(-1)
    out = _gather_kernel(flat_idx, embedding)
    return out.reshape(_BATCH, _FIELDS, _EMBED_DIM)


# speedup vs baseline: 1.5774x; 1.0201x over previous
import functools

import jax
import jax.numpy as jnp
from jax import lax
from jax.experimental import pallas as pl
from jax.experimental.pallas import tpu as pltpu
from jax.experimental.pallas import tpu_sc as plsc

_VOCAB = 1000000
_EMBED_DIM = 32
_BATCH = 16384
_FIELDS = 26

_N_TOTAL = _BATCH * _FIELDS
_NC, _NS = 2, 16
_NW = _NC * _NS
_PER_W = _N_TOTAL // _NW
_CHUNK = 1664
_N_CHUNKS = _PER_W // _CHUNK

_mesh = plsc.VectorSubcoreMesh(core_axis_name="c", subcore_axis_name="s")


@functools.partial(
    pl.kernel,
    mesh=_mesh,
    out_type=jax.ShapeDtypeStruct((_N_TOTAL, _EMBED_DIM), jnp.float32),
    scratch_types=[
        pltpu.VMEM((_PER_W,), jnp.int32),
        pltpu.VMEM((2, _CHUNK, _EMBED_DIM), jnp.float32),
        pltpu.SemaphoreType.DMA,
        pltpu.SemaphoreType.DMA,
        pltpu.SemaphoreType.DMA,
        pltpu.SemaphoreType.DMA,
    ],
    compiler_params=pltpu.CompilerParams(use_tc_tiling_on_sc=False),
)
def _gather_kernel(idx_hbm, table_hbm, out_hbm, idx_v, rows_v, sg0, sg1,
                   ss0, ss1):
    wid = lax.axis_index("s") * _NC + lax.axis_index("c")
    base = wid * _PER_W
    sem_g = (sg0, sg1)
    sem_s = (ss0, ss1)

    pltpu.sync_copy(idx_hbm.at[pl.ds(base, _PER_W)], idx_v)

    gathers = [None] * _N_CHUNKS
    stores = [None] * _N_CHUNKS

    def start_gather(c):
        slot = c & 1
        g = pltpu.make_async_copy(
            table_hbm.at[idx_v.at[pl.ds(c * _CHUNK, _CHUNK)]],
            rows_v.at[slot], sem_g[slot])
        g.start()
        gathers[c] = g

    def start_store(c):
        slot = c & 1
        s = pltpu.make_async_copy(
            rows_v.at[slot],
            out_hbm.at[pl.ds(base + c * _CHUNK, _CHUNK)], sem_s[slot])
        s.start()
        stores[c] = s

    for c in range(_N_CHUNKS):
        if c >= 2:
            stores[c - 2].wait()
        start_gather(c)
        if c >= 1:
            gathers[c - 1].wait()
            start_store(c - 1)
    gathers[_N_CHUNKS - 1].wait()
    start_store(_N_CHUNKS - 1)
    stores[_N_CHUNKS - 2].wait()
    stores[_N_CHUNKS - 1].wait()


def kernel(inputs, embedding):
    flat_idx = inputs.reshape(-1)
    out = _gather_kernel(flat_idx, embedding)
    return out.reshape(_BATCH, _FIELDS, _EMBED_DIM)


# speedup vs baseline: 1.6712x; 1.0594x over previous
import functools

import jax
import jax.numpy as jnp
from jax import lax
from jax.experimental import pallas as pl
from jax.experimental.pallas import tpu as pltpu
from jax.experimental.pallas import tpu_sc as plsc

_VOCAB = 1000000
_EMBED_DIM = 32
_BATCH = 16384
_FIELDS = 26

_N_TOTAL = _BATCH * _FIELDS
_NC, _NS = 2, 16
_NW = _NC * _NS
_B_PER_W = _BATCH // _NW
_PER_W = _B_PER_W * _FIELDS
_B_CHUNK = 64
_CHUNK = _B_CHUNK * _FIELDS
_N_CHUNKS = _B_PER_W // _B_CHUNK

_mesh = plsc.VectorSubcoreMesh(core_axis_name="c", subcore_axis_name="s")


@functools.partial(
    pl.kernel,
    mesh=_mesh,
    out_type=jax.ShapeDtypeStruct((_N_TOTAL, _EMBED_DIM), jnp.float32),
    scratch_types=[
        pltpu.VMEM((_PER_W,), jnp.int32),
        pltpu.VMEM((2, _CHUNK, _EMBED_DIM), jnp.float32),
        pltpu.SemaphoreType.DMA,
        pltpu.SemaphoreType.DMA,
        pltpu.SemaphoreType.DMA,
        pltpu.SemaphoreType.DMA,
    ],
    compiler_params=pltpu.CompilerParams(use_tc_tiling_on_sc=False),
)
def _gather_kernel(idx_hbm, table_hbm, out_hbm, idx_v, rows_v, sg0, sg1,
                   ss0, ss1):
    wid = lax.axis_index("s") * _NC + lax.axis_index("c")
    base = wid * _PER_W
    sem_g = (sg0, sg1)
    sem_s = (ss0, ss1)

    pltpu.sync_copy(idx_hbm.at[pl.ds(base, _PER_W)], idx_v)

    gathers = [None] * _N_CHUNKS
    stores = [None] * _N_CHUNKS

    def start_gather(c):
        slot = c & 1
        g = pltpu.make_async_copy(
            table_hbm.at[idx_v.at[pl.ds(c * _CHUNK, _CHUNK)]],
            rows_v.at[slot], sem_g[slot])
        g.start()
        gathers[c] = g

    def start_store(c):
        slot = c & 1
        s = pltpu.make_async_copy(
            rows_v.at[slot],
            out_hbm.at[pl.ds(base + c * _CHUNK, _CHUNK)], sem_s[slot])
        s.start()
        stores[c] = s

    for c in range(_N_CHUNKS):
        if c >= 2:
            stores[c - 2].wait()
        start_gather(c)
        if c >= 1:
            gathers[c - 1].wait()
            start_store(c - 1)
    gathers[_N_CHUNKS - 1].wait()
    start_store(_N_CHUNKS - 1)
    stores[_N_CHUNKS - 2].wait()
    stores[_N_CHUNKS - 1].wait()


def kernel(inputs, embedding):
    flat_idx = inputs.T.reshape(-1)
    out = _gather_kernel(flat_idx, embedding)
    return out.reshape(_FIELDS, _BATCH, _EMBED_DIM).transpose(1, 0, 2)
